# baseline (speedup 1.0000x reference)
import functools

import jax
import jax.numpy as jnp
import numpy as np
from jax.experimental import pallas as pl
from jax.experimental.pallas import tpu as pltpu
from jax.experimental.shard_map import shard_map
from jax.sharding import Mesh, PartitionSpec as P

_B, _D, _T = 32, 64, 576
_K = 1024
_COMMITMENT_COST = 0.25
_VQ_COST = 1.0


def _vq_kernel(x_ref, e_ref, q_ref, idx_ref, sse_ref, enorm_ref):
    b = pl.program_id(0)
    xb = x_ref[0]
    emb = e_ref[...]

    @pl.when(b == 0)
    def _first():
        enorm_ref[...] = jnp.sum(emb * emb, axis=1, keepdims=True)
        sse_ref[...] = jnp.zeros((1, 1), jnp.float32)

    g = jax.lax.dot_general(emb, xb, (((1,), (0,)), ((), ())),
                            preferred_element_type=jnp.float32)
    score = enorm_ref[...] - 2.0 * g
    idx = jnp.argmin(score, axis=0).astype(jnp.int32)
    idx_ref[0, 0] = idx
    iota_k = jax.lax.broadcasted_iota(jnp.int32, (_K, _T), 0)
    onehot = (iota_k == idx[None, :]).astype(jnp.float32)
    qb = jax.lax.dot_general(emb, onehot, (((0,), (0,)), ((), ())),
                             preferred_element_type=jnp.float32)
    q_ref[0] = qb
    diff = xb - qb
    sse_ref[...] += jnp.sum(diff * diff).reshape(1, 1)


def _run_shard(xs, emb, nb):
    return pl.pallas_call(
        _vq_kernel,
        grid=(nb,),
        in_specs=[
            pl.BlockSpec((1, _D, _T), lambda b: (b, 0, 0)),
            pl.BlockSpec((_K, _D), lambda b: (0, 0)),
        ],
        out_specs=[
            pl.BlockSpec((1, _D, _T), lambda b: (b, 0, 0)),
            pl.BlockSpec((1, 1, _T), lambda b: (b, 0, 0)),
            pl.BlockSpec((1, 1), lambda b: (0, 0)),
        ],
        out_shape=[
            jax.ShapeDtypeStruct((nb, _D, _T), jnp.float32),
            jax.ShapeDtypeStruct((nb, 1, _T), jnp.int32),
            jax.ShapeDtypeStruct((1, 1), jnp.float32),
        ],
        scratch_shapes=[pltpu.VMEM((_K, 1), jnp.float32)],
    )(xs, emb)


def kernel(x, embeddings):
    devs = jax.devices()
    n = min(len(devs), _B)
    while _B % n:
        n -= 1
    if n > 1:
        mesh = Mesh(np.array(devs[:n]), ("b",))

        @functools.partial(
            shard_map, mesh=mesh,
            in_specs=(P("b"), P()),
            out_specs=(P("b"), P("b"), P("b")),
            check_rep=False,
        )
        def run(xs, emb):
            q, idx, sse = _run_shard(xs, emb, _B // n)
            return q, idx, sse

        q, idx, sse = run(x, embeddings)
    else:
        q, idx, sse = _run_shard(x, embeddings, _B)
    e = jnp.sum(sse) / (_B * _T * _D)
    loss_commit = _COMMITMENT_COST * e
    loss_vq = _VQ_COST * e
    return q, loss_commit, loss_vq, idx.reshape(_B * _T)


# speedup vs baseline: 9.9667x; 9.9667x over previous
import jax
import jax.numpy as jnp
from jax.experimental import pallas as pl
from jax.experimental.pallas import tpu as pltpu

_B, _D, _T = 32, 64, 576
_K = 1024
_COMMITMENT_COST = 0.25
_VQ_COST = 1.0


def _vq_kernel(x_ref, e_ref, q_ref, idx_ref, sse_ref, enorm_ref, eaug_ref):
    b = pl.program_id(0)
    emb = e_ref[...]

    @pl.when(b == 0)
    def _first():
        enorm_ref[...] = jnp.sum(emb * emb, axis=1, keepdims=True)
        k = jax.lax.broadcasted_iota(jnp.int32, (_K, 1), 0)
        khi = ((k >> 5) << 5).astype(jnp.float32)
        klo = (k & 31).astype(jnp.float32)
        eaug_ref[...] = jnp.concatenate([emb, khi, klo], axis=1)
        sse_ref[...] = jnp.zeros((1, 1), jnp.float32)

    xb = x_ref[0]
    g = jax.lax.dot_general(emb, xb, (((1,), (0,)), ((), ())),
                            preferred_element_type=jnp.float32)
    score = enorm_ref[...] - 2.0 * g
    minv = jnp.min(score, axis=0)
    onehot = jnp.where(score == minv[None, :], 1.0, 0.0)
    qa = jax.lax.dot_general(eaug_ref[...], onehot, (((0,), (0,)), ((), ())),
                             preferred_element_type=jnp.float32)
    qb = qa[:_D]
    q_ref[0] = qb
    idx_ref[0, 0] = (qa[_D] + qa[_D + 1]).astype(jnp.int32)
    diff = xb - qb
    sse_ref[...] += jnp.sum(diff * diff).reshape(1, 1)


def kernel(x, embeddings):
    q, idx, sse = pl.pallas_call(
        _vq_kernel,
        grid=(_B,),
        in_specs=[
            pl.BlockSpec((1, _D, _T), lambda b: (b, 0, 0)),
            pl.BlockSpec((_K, _D), lambda b: (0, 0)),
        ],
        out_specs=[
            pl.BlockSpec((1, _D, _T), lambda b: (b, 0, 0)),
            pl.BlockSpec((1, 1, _T), lambda b: (b, 0, 0)),
            pl.BlockSpec((1, 1), lambda b: (0, 0)),
        ],
        out_shape=[
            jax.ShapeDtypeStruct((_B, _D, _T), jnp.float32),
            jax.ShapeDtypeStruct((_B, 1, _T), jnp.int32),
            jax.ShapeDtypeStruct((1, 1), jnp.float32),
        ],
        scratch_shapes=[
            pltpu.VMEM((_K, 1), jnp.float32),
            pltpu.VMEM((_K, _D + 2), jnp.float32),
        ],
    )(x, embeddings)
    e = sse[0, 0] / (_B * _T * _D)
    loss_commit = _COMMITMENT_COST * e
    loss_vq = _VQ_COST * e
    return q, loss_commit, loss_vq, idx.reshape(_B * _T)


# speedup vs baseline: 10.2762x; 1.0310x over previous
import jax
import jax.numpy as jnp
from jax.experimental import pallas as pl
from jax.experimental.pallas import tpu as pltpu

_B, _D, _T = 32, 64, 576
_K = 1024
_COMMITMENT_COST = 0.25
_VQ_COST = 1.0
_STEPS = _B // 2 + 1


def _min_onehot(score):
    minv = jnp.min(score, axis=0)
    return minv, jnp.where(score == minv[None, :], 1.0, 0.0)


def _vq_kernel(x_ref, e_ref, q_ref, idx_ref, sse_ref,
               enorm_ref, eaug_ref, g0_ref, g1_ref, oh_ref):
    s = pl.program_id(0)
    emb = e_ref[...]

    @pl.when(s == 0)
    def _first():
        enorm_ref[...] = jnp.sum(emb * emb, axis=1, keepdims=True)
        k = jax.lax.broadcasted_iota(jnp.int32, (_K, 1), 0)
        khi = ((k >> 5) << 5).astype(jnp.float32)
        klo = (k & 31).astype(jnp.float32)
        eaug_ref[...] = jnp.concatenate([emb, khi, klo], axis=1)
        sse_ref[...] = jnp.zeros((1, 1), jnp.float32)

    enorm = enorm_ref[...]
    eaug = eaug_ref[...]
    dot_kk = (((0,), (0,)), ((), ()))
    dot_kd = (((1,), (0,)), ((), ()))

    minv1, oh1 = _min_onehot(enorm - 2.0 * g1_ref[...])
    qa_e = jax.lax.dot_general(eaug, oh_ref[...], dot_kk,
                               preferred_element_type=jnp.float32)
    q_ref[0] = qa_e[:_D]
    idx_ref[0, 0] = (qa_e[_D] + qa_e[_D + 1]).astype(jnp.int32)
    g0_ref[...] = jax.lax.dot_general(emb, x_ref[0], dot_kd,
                                      preferred_element_type=jnp.float32)
    g1_ref[...] = jax.lax.dot_general(emb, x_ref[1], dot_kd,
                                      preferred_element_type=jnp.float32)
    minv0, oh0 = _min_onehot(enorm - 2.0 * g0_ref[...])
    oh_ref[...] = oh0
    qa_o = jax.lax.dot_general(eaug, oh1, dot_kk,
                               preferred_element_type=jnp.float32)
    q_ref[1] = qa_o[:_D]
    idx_ref[1, 0] = (qa_o[_D] + qa_o[_D + 1]).astype(jnp.int32)

    xb = x_ref[...]
    sse_ref[...] += (
        jnp.where(s < _STEPS - 1, jnp.sum(xb * xb) + jnp.sum(minv0), 0.0)
        + jnp.where(s >= 1, jnp.sum(minv1), 0.0)
    ).reshape(1, 1)


def kernel(x, embeddings):
    q, idx, sse = pl.pallas_call(
        _vq_kernel,
        grid=(_STEPS,),
        in_specs=[
            pl.BlockSpec((2, _D, _T), lambda s: (jnp.minimum(s, _STEPS - 2), 0, 0)),
            pl.BlockSpec((_K, _D), lambda s: (0, 0)),
        ],
        out_specs=[
            pl.BlockSpec((2, _D, _T), lambda s: (jnp.maximum(s - 1, 0), 0, 0)),
            pl.BlockSpec((2, 1, _T), lambda s: (jnp.maximum(s - 1, 0), 0, 0)),
            pl.BlockSpec((1, 1), lambda s: (0, 0)),
        ],
        out_shape=[
            jax.ShapeDtypeStruct((_B, _D, _T), jnp.float32),
            jax.ShapeDtypeStruct((_B, 1, _T), jnp.int32),
            jax.ShapeDtypeStruct((1, 1), jnp.float32),
        ],
        scratch_shapes=[
            pltpu.VMEM((_K, 1), jnp.float32),
            pltpu.VMEM((_K, _D + 2), jnp.float32),
            pltpu.VMEM((_K, _T), jnp.float32),
            pltpu.VMEM((_K, _T), jnp.float32),
            pltpu.VMEM((_K, _T), jnp.float32),
        ],
    )(x, embeddings)
    e = sse[0, 0] / (_B * _T * _D)
    loss_commit = _COMMITMENT_COST * e
    loss_vq = _VQ_COST * e
    return q, loss_commit, loss_vq, idx.reshape(_B * _T)
